# Initial kernel scaffold; baseline (speedup 1.0000x reference)
import jax
import jax.numpy as jnp
from jax.experimental import pallas as pl


def kernel(arm_locs, arm_scores, odm_locs, odm_scores, boxes, labels, priors_cxcy):
    raise NotImplementedError("write your pallas kernel here")


# speedup vs baseline: 15.5223x; 15.5223x over previous
import jax
import jax.numpy as jnp
from jax import lax
from jax.experimental import pallas as pl
from jax.experimental.pallas import tpu as pltpu

_B, _P, _NOBJ, _NC = 16, 16320, 12, 21
_R, _C = 8, 2040
_THRESHOLD, _NEG_POS_RATIO, _THETA, _ALPHA = 0.5, 3, 0.01, 1.0


def _flat_idx():
    r = lax.broadcasted_iota(jnp.int32, (_R, _C), 0)
    c = lax.broadcasted_iota(jnp.int32, (_R, _C), 1)
    return r * _C + c


def _match(boxes_ref, labels_ref, ax1, ay1, ax2, ay2, pcx, pcy, pw, ph):
    area_b = (ax2 - ax1) * (ay2 - ay1)
    fidx = _flat_idx()
    best = None
    obj = None
    mxs, pfs = [], []
    for j in range(_NOBJ):
        bx1 = boxes_ref[0, j, 0]
        by1 = boxes_ref[0, j, 1]
        bx2 = boxes_ref[0, j, 2]
        by2 = boxes_ref[0, j, 3]
        w = jnp.maximum(jnp.minimum(bx2, ax2) - jnp.maximum(bx1, ax1), 0.0)
        h = jnp.maximum(jnp.minimum(by2, ay2) - jnp.maximum(by1, ay1), 0.0)
        inter = w * h
        area_a = (bx2 - bx1) * (by2 - by1)
        iou = inter / (area_a + area_b - inter)
        mx = jnp.max(iou)
        pfs.append(jnp.min(jnp.where(iou == mx, fidx, _P)))
        mxs.append(mx)
        if j == 0:
            best = iou
            obj = jnp.zeros((_R, _C), jnp.int32)
        else:
            obj = jnp.where(iou > best, j, obj)
            best = jnp.maximum(best, iou)
    rank = jnp.int32(-1)
    for j in range(_NOBJ):
        valid = mxs[j] > 0.0
        rank = rank + valid.astype(jnp.int32)
        m = jnp.logical_and(valid, fidx == pfs[j])
        best = jnp.where(m, 1.0, best)
        obj = jnp.where(m, rank, obj)
    lab = jnp.zeros((_R, _C), jnp.int32)
    gx1 = jnp.zeros((_R, _C), jnp.float32)
    gy1 = jnp.zeros((_R, _C), jnp.float32)
    gx2 = jnp.zeros((_R, _C), jnp.float32)
    gy2 = jnp.zeros((_R, _C), jnp.float32)
    for j in range(_NOBJ):
        sel = obj == j
        lab = jnp.where(sel, labels_ref[0, 0, j], lab)
        gx1 = jnp.where(sel, boxes_ref[0, j, 0], gx1)
        gy1 = jnp.where(sel, boxes_ref[0, j, 1], gy1)
        gx2 = jnp.where(sel, boxes_ref[0, j, 2], gx2)
        gy2 = jnp.where(sel, boxes_ref[0, j, 3], gy2)
    lab = jnp.where(best < _THRESHOLD, 0, lab)
    cx = (gx1 + gx2) / 2.0
    cy = (gy1 + gy2) / 2.0
    w = gx2 - gx1
    h = gy2 - gy1
    t0 = (cx - pcx) / (pw / 10.0)
    t1 = (cy - pcy) / (ph / 10.0)
    t2 = jnp.log(w / pw) * 5.0
    t3 = jnp.log(h / ph) * 5.0
    return lab, (t0, t1, t2, t3)


def _topk_sum(neg, k):
    vi = lax.bitcast_convert_type(neg, jnp.int32)

    def body(_, lohi):
        lo, hi = lohi
        mid = lo + (hi - lo + 1) // 2
        cnt = jnp.sum((vi >= mid).astype(jnp.int32))
        ok = cnt >= k
        return jnp.where(ok, mid, lo), jnp.where(ok, hi, mid - 1)

    lo, _ = lax.fori_loop(0, 31, body, (jnp.int32(0), jnp.int32(0x7F800000)))
    tf = jnp.max(lax.bitcast_convert_type(jnp.broadcast_to(lo, (_R, _C)), jnp.float32))
    s = jnp.sum(jnp.maximum(neg - tf, 0.0))
    return jnp.where(k > 0, k.astype(jnp.float32) * tf + s, 0.0)


def _loc_loss_sum(pred, tgt, posf):
    acc = jnp.float32(0.0)
    for c in range(4):
        d = jnp.abs(pred[c] - tgt[c])
        acc = acc + jnp.sum(jnp.where(d < 1.0, 0.5 * d * d, d - 0.5) * posf)
    return acc


def _body(pr_ref, boxes_ref, labels_ref, al_ref, as_ref, ol_ref, os_ref,
          out_ref, acc_ref):
    i = pl.program_id(0)
    pcx = pr_ref[0]
    pcy = pr_ref[1]
    pw = pr_ref[2]
    ph = pr_ref[3]
    px1 = pcx - pw / 2.0
    py1 = pcy - ph / 2.0
    px2 = pcx + pw / 2.0
    py2 = pcy + ph / 2.0

    lab_a, ta = _match(boxes_ref, labels_ref, px1, py1, px2, py2,
                       pcx, pcy, pw, ph)
    pos_a = lab_a > 0
    posf_a = pos_a.astype(jnp.float32)
    n_pos_a = jnp.sum(posf_a)
    al = [al_ref[0, c] for c in range(4)]
    loc_a = _loc_loss_sum(al, ta, posf_a)
    s0 = as_ref[0, 0]
    s1 = as_ref[0, 1]
    m2 = jnp.maximum(s0, s1)
    lse2 = m2 + jnp.log(jnp.exp(s0 - m2) + jnp.exp(s1 - m2))
    ce_a = lse2 - jnp.where(pos_a, s1, s0)
    cpos_a = jnp.sum(ce_a * posf_a)
    neg_a = jnp.where(pos_a, 0.0, ce_a)
    k_a = _NEG_POS_RATIO * jnp.sum(pos_a.astype(jnp.int32))
    hard_a = _topk_sum(neg_a, k_a)

    a0 = al[0]
    a1 = al[1]
    a2 = al[2]
    a3 = al[3]
    dcx = a0 * pw / 10.0 + pcx
    dcy = a1 * ph / 10.0 + pcy
    dw = jnp.exp(a2 / 5.0) * pw
    dh = jnp.exp(a3 / 5.0) * ph
    dx1 = dcx - dw / 2.0
    dy1 = dcy - dh / 2.0
    dx2 = dcx + dw / 2.0
    dy2 = dcy + dh / 2.0
    ocx = (dx2 + dx1) / 2.0
    ocy = (dy2 + dy1) / 2.0
    ow = dx2 - dx1
    oh = dy2 - dy1
    lab_o, to = _match(boxes_ref, labels_ref, dx1, dy1, dx2, dy2,
                       ocx, ocy, ow, oh)
    em = jnp.maximum(s0, s1)
    e0 = jnp.exp(s0 - em)
    e1 = jnp.exp(s1 - em)
    easy = e1 / (e0 + e1) < _THETA
    pos_o = jnp.logical_and(lab_o > 0, jnp.logical_not(easy))
    posf_o = pos_o.astype(jnp.float32)
    n_pos_o = jnp.sum(posf_o)
    ol = [ol_ref[0, c] for c in range(4)]
    loc_o = _loc_loss_sum(ol, to, posf_o)
    sc = [os_ref[0, c] for c in range(_NC)]
    mo = sc[0]
    for c in range(1, _NC):
        mo = jnp.maximum(mo, sc[c])
    se = jnp.zeros((_R, _C), jnp.float32)
    st = jnp.zeros((_R, _C), jnp.float32)
    for c in range(_NC):
        se = se + jnp.exp(sc[c] - mo)
        st = st + jnp.where(lab_o == c, sc[c], 0.0)
    ce_o = (mo + jnp.log(se)) - st
    cpos_o = jnp.sum(ce_o * posf_o)
    neg_o = jnp.where(pos_o, 0.0, ce_o)
    neg_o = jnp.where(easy, 0.0, neg_o)
    k_o = _NEG_POS_RATIO * jnp.sum(pos_o.astype(jnp.int32))
    hard_o = _topk_sum(neg_o, k_o)

    @pl.when(i == 0)
    def _init():
        for t in range(8):
            acc_ref[t] = 0.0

    parts = (loc_a, cpos_a, hard_a, n_pos_a, loc_o, cpos_o, hard_o, n_pos_o)
    for t, v in enumerate(parts):
        acc_ref[t] = acc_ref[t] + v

    @pl.when(i == _B - 1)
    def _fin():
        na = acc_ref[3]
        no = acc_ref[7]
        arm = (acc_ref[2] + acc_ref[1]) / na + _ALPHA * acc_ref[0] / (na * 4.0)
        odm = (acc_ref[6] + acc_ref[5]) / no + _ALPHA * acc_ref[4] / (no * 4.0)
        out_ref[0, 0] = arm + odm


def kernel(arm_locs, arm_scores, odm_locs, odm_scores, boxes, labels,
           priors_cxcy):
    al = arm_locs.transpose(0, 2, 1).reshape(_B, 4, _R, _C)
    asr = arm_scores.transpose(0, 2, 1).reshape(_B, 2, _R, _C)
    ol = odm_locs.transpose(0, 2, 1).reshape(_B, 4, _R, _C)
    osr = odm_scores.transpose(0, 2, 1).reshape(_B, _NC, _R, _C)
    pr = priors_cxcy.T.reshape(4, _R, _C)
    out = pl.pallas_call(
        _body,
        grid=(_B,),
        in_specs=[
            pl.BlockSpec((4, _R, _C), lambda i: (0, 0, 0)),
            pl.BlockSpec((1, _NOBJ, 4), lambda i: (i, 0, 0),
                         memory_space=pltpu.SMEM),
            pl.BlockSpec((1, 1, _NOBJ), lambda i: (i, 0, 0),
                         memory_space=pltpu.SMEM),
            pl.BlockSpec((1, 4, _R, _C), lambda i: (i, 0, 0, 0)),
            pl.BlockSpec((1, 2, _R, _C), lambda i: (i, 0, 0, 0)),
            pl.BlockSpec((1, 4, _R, _C), lambda i: (i, 0, 0, 0)),
            pl.BlockSpec((1, _NC, _R, _C), lambda i: (i, 0, 0, 0)),
        ],
        out_specs=pl.BlockSpec((1, 1), lambda i: (0, 0),
                               memory_space=pltpu.SMEM),
        out_shape=jax.ShapeDtypeStruct((1, 1), jnp.float32),
        scratch_shapes=[pltpu.SMEM((8,), jnp.float32)],
    )(pr, boxes, labels.astype(jnp.int32).reshape(_B, 1, _NOBJ),
      al, asr, ol, osr)
    return out[0, 0]


# speedup vs baseline: 24.5854x; 1.5839x over previous
import jax
import jax.numpy as jnp
from jax import lax
from jax.experimental import pallas as pl
from jax.experimental.pallas import tpu as pltpu

_B, _P, _NOBJ, _NC = 16, 16320, 12, 21
_R, _C = 8, 2040
_THRESHOLD, _NEG_POS_RATIO, _THETA, _ALPHA = 0.5, 3, 0.01, 1.0


def _flat_idx():
    r = lax.broadcasted_iota(jnp.int32, (_R, _C), 0)
    c = lax.broadcasted_iota(jnp.int32, (_R, _C), 1)
    return r * _C + c


def _match(boxes_ref, labels_ref, ax1, ay1, ax2, ay2, pcx, pcy, pw, ph):
    area_b = (ax2 - ax1) * (ay2 - ay1)
    fidx = _flat_idx()
    best = None
    obj = None
    mxs, pfs = [], []
    for j in range(_NOBJ):
        bx1 = boxes_ref[0, j, 0]
        by1 = boxes_ref[0, j, 1]
        bx2 = boxes_ref[0, j, 2]
        by2 = boxes_ref[0, j, 3]
        w = jnp.maximum(jnp.minimum(bx2, ax2) - jnp.maximum(bx1, ax1), 0.0)
        h = jnp.maximum(jnp.minimum(by2, ay2) - jnp.maximum(by1, ay1), 0.0)
        inter = w * h
        area_a = (bx2 - bx1) * (by2 - by1)
        iou = inter / (area_a + area_b - inter)
        mx = jnp.max(iou)
        pfs.append(jnp.min(jnp.where(iou == mx, fidx, _P)))
        mxs.append(mx)
        if j == 0:
            best = iou
            obj = jnp.zeros((_R, _C), jnp.int32)
        else:
            obj = jnp.where(iou > best, j, obj)
            best = jnp.maximum(best, iou)
    rank = jnp.int32(-1)
    for j in range(_NOBJ):
        valid = mxs[j] > 0.0
        rank = rank + valid.astype(jnp.int32)
        m = jnp.logical_and(valid, fidx == pfs[j])
        best = jnp.where(m, 1.0, best)
        obj = jnp.where(m, rank, obj)
    lab = jnp.zeros((_R, _C), jnp.int32)
    gx1 = jnp.zeros((_R, _C), jnp.float32)
    gy1 = jnp.zeros((_R, _C), jnp.float32)
    gx2 = jnp.zeros((_R, _C), jnp.float32)
    gy2 = jnp.zeros((_R, _C), jnp.float32)
    for j in range(_NOBJ):
        sel = obj == j
        lab = jnp.where(sel, labels_ref[0, 0, j], lab)
        gx1 = jnp.where(sel, boxes_ref[0, j, 0], gx1)
        gy1 = jnp.where(sel, boxes_ref[0, j, 1], gy1)
        gx2 = jnp.where(sel, boxes_ref[0, j, 2], gx2)
        gy2 = jnp.where(sel, boxes_ref[0, j, 3], gy2)
    lab = jnp.where(best < _THRESHOLD, 0, lab)
    cx = (gx1 + gx2) / 2.0
    cy = (gy1 + gy2) / 2.0
    w = gx2 - gx1
    h = gy2 - gy1
    t0 = (cx - pcx) / (pw / 10.0)
    t1 = (cy - pcy) / (ph / 10.0)
    t2 = jnp.log(w / pw) * 5.0
    t3 = jnp.log(h / ph) * 5.0
    return lab, (t0, t1, t2, t3)


_C2 = 2048
_NROW = 2 * _B


def _batched_topk_sums(neg_ref, kv_ref):
    kcol = kv_ref[:, 0:1]

    def count_ge(mid):
        cnt = jnp.zeros((_NROW, 128), jnp.int32)
        m3 = mid[:, None, :]
        for c in range(_C2 // 128):
            blk = lax.bitcast_convert_type(
                neg_ref[:, :, pl.ds(c * 128, 128)], jnp.int32)
            cnt = cnt + jnp.sum((blk >= m3).astype(jnp.int32), axis=1)
        return jnp.sum(cnt, axis=1, keepdims=True)

    def body(_, lohi):
        lo, hi = lohi
        mid = lo + (hi - lo + 1) // 2
        ok = count_ge(mid) >= kcol
        return jnp.where(ok, mid, lo), jnp.where(ok, hi, mid - 1)

    lo, _ = lax.fori_loop(
        0, 31, body,
        (jnp.zeros((_NROW, 1), jnp.int32),
         jnp.full((_NROW, 1), 0x7F800000, jnp.int32)))
    tf = lax.bitcast_convert_type(lo, jnp.float32)
    srel = jnp.zeros((_NROW, 128), jnp.float32)
    t3 = tf[:, None, :]
    for c in range(_C2 // 128):
        blk = neg_ref[:, :, pl.ds(c * 128, 128)]
        srel = srel + jnp.sum(jnp.maximum(blk - t3, 0.0), axis=1)
    s = jnp.sum(srel, axis=1, keepdims=True)
    kf = kcol.astype(jnp.float32)
    hard = jnp.where(kcol > 0, kf * tf + s, 0.0)
    return jnp.sum(hard[:_B, 0]), jnp.sum(hard[_B:, 0])


def _loc_loss_sum(pred, tgt, posf):
    acc = jnp.float32(0.0)
    for c in range(4):
        d = jnp.abs(pred[c] - tgt[c])
        acc = acc + jnp.sum(jnp.where(d < 1.0, 0.5 * d * d, d - 0.5) * posf)
    return acc


def _body(pr_ref, boxes_ref, labels_ref, al_ref, as_ref, ol_ref, os_ref,
          out_ref, acc_ref, neg_ref, kv_ref):
    i = pl.program_id(0)

    @pl.when(i == 0)
    def _zero():
        neg_ref[...] = jnp.zeros((_NROW, _R, _C2), jnp.float32)

    pcx = pr_ref[0]
    pcy = pr_ref[1]
    pw = pr_ref[2]
    ph = pr_ref[3]
    px1 = pcx - pw / 2.0
    py1 = pcy - ph / 2.0
    px2 = pcx + pw / 2.0
    py2 = pcy + ph / 2.0

    lab_a, ta = _match(boxes_ref, labels_ref, px1, py1, px2, py2,
                       pcx, pcy, pw, ph)
    pos_a = lab_a > 0
    posf_a = pos_a.astype(jnp.float32)
    n_pos_a = jnp.sum(posf_a)
    al = [al_ref[0, c] for c in range(4)]
    loc_a = _loc_loss_sum(al, ta, posf_a)
    s0 = as_ref[0, 0]
    s1 = as_ref[0, 1]
    m2 = jnp.maximum(s0, s1)
    lse2 = m2 + jnp.log(jnp.exp(s0 - m2) + jnp.exp(s1 - m2))
    ce_a = lse2 - jnp.where(pos_a, s1, s0)
    cpos_a = jnp.sum(ce_a * posf_a)
    neg_a = jnp.where(pos_a, 0.0, ce_a)
    k_a = _NEG_POS_RATIO * jnp.sum(pos_a.astype(jnp.int32))
    neg_ref[i, :, pl.ds(0, _C)] = neg_a
    kv_ref[pl.ds(i, 1), :] = jnp.full((1, 128), k_a, jnp.int32)

    a0 = al[0]
    a1 = al[1]
    a2 = al[2]
    a3 = al[3]
    dcx = a0 * pw / 10.0 + pcx
    dcy = a1 * ph / 10.0 + pcy
    dw = jnp.exp(a2 / 5.0) * pw
    dh = jnp.exp(a3 / 5.0) * ph
    dx1 = dcx - dw / 2.0
    dy1 = dcy - dh / 2.0
    dx2 = dcx + dw / 2.0
    dy2 = dcy + dh / 2.0
    ocx = (dx2 + dx1) / 2.0
    ocy = (dy2 + dy1) / 2.0
    ow = dx2 - dx1
    oh = dy2 - dy1
    lab_o, to = _match(boxes_ref, labels_ref, dx1, dy1, dx2, dy2,
                       ocx, ocy, ow, oh)
    em = jnp.maximum(s0, s1)
    e0 = jnp.exp(s0 - em)
    e1 = jnp.exp(s1 - em)
    easy = e1 / (e0 + e1) < _THETA
    pos_o = jnp.logical_and(lab_o > 0, jnp.logical_not(easy))
    posf_o = pos_o.astype(jnp.float32)
    n_pos_o = jnp.sum(posf_o)
    ol = [ol_ref[0, c] for c in range(4)]
    loc_o = _loc_loss_sum(ol, to, posf_o)
    sc = [os_ref[0, c] for c in range(_NC)]
    mo = sc[0]
    for c in range(1, _NC):
        mo = jnp.maximum(mo, sc[c])
    se = jnp.zeros((_R, _C), jnp.float32)
    st = jnp.zeros((_R, _C), jnp.float32)
    for c in range(_NC):
        se = se + jnp.exp(sc[c] - mo)
        st = st + jnp.where(lab_o == c, sc[c], 0.0)
    ce_o = (mo + jnp.log(se)) - st
    cpos_o = jnp.sum(ce_o * posf_o)
    neg_o = jnp.where(pos_o, 0.0, ce_o)
    neg_o = jnp.where(easy, 0.0, neg_o)
    k_o = _NEG_POS_RATIO * jnp.sum(pos_o.astype(jnp.int32))
    neg_ref[_B + i, :, pl.ds(0, _C)] = neg_o
    kv_ref[pl.ds(_B + i, 1), :] = jnp.full((1, 128), k_o, jnp.int32)

    @pl.when(i == 0)
    def _init():
        for t in range(6):
            acc_ref[t] = 0.0

    parts = (loc_a, cpos_a, n_pos_a, loc_o, cpos_o, n_pos_o)
    for t, v in enumerate(parts):
        acc_ref[t] = acc_ref[t] + v

    @pl.when(i == _B - 1)
    def _fin():
        hard_a, hard_o = _batched_topk_sums(neg_ref, kv_ref)
        na = acc_ref[2]
        no = acc_ref[5]
        arm = (hard_a + acc_ref[1]) / na + _ALPHA * acc_ref[0] / (na * 4.0)
        odm = (hard_o + acc_ref[4]) / no + _ALPHA * acc_ref[3] / (no * 4.0)
        out_ref[0, 0] = arm + odm


def kernel(arm_locs, arm_scores, odm_locs, odm_scores, boxes, labels,
           priors_cxcy):
    al = arm_locs.transpose(0, 2, 1).reshape(_B, 4, _R, _C)
    asr = arm_scores.transpose(0, 2, 1).reshape(_B, 2, _R, _C)
    ol = odm_locs.transpose(0, 2, 1).reshape(_B, 4, _R, _C)
    osr = odm_scores.transpose(0, 2, 1).reshape(_B, _NC, _R, _C)
    pr = priors_cxcy.T.reshape(4, _R, _C)
    out = pl.pallas_call(
        _body,
        grid=(_B,),
        in_specs=[
            pl.BlockSpec((4, _R, _C), lambda i: (0, 0, 0)),
            pl.BlockSpec((1, _NOBJ, 4), lambda i: (i, 0, 0),
                         memory_space=pltpu.SMEM),
            pl.BlockSpec((1, 1, _NOBJ), lambda i: (i, 0, 0),
                         memory_space=pltpu.SMEM),
            pl.BlockSpec((1, 4, _R, _C), lambda i: (i, 0, 0, 0)),
            pl.BlockSpec((1, 2, _R, _C), lambda i: (i, 0, 0, 0)),
            pl.BlockSpec((1, 4, _R, _C), lambda i: (i, 0, 0, 0)),
            pl.BlockSpec((1, _NC, _R, _C), lambda i: (i, 0, 0, 0)),
        ],
        out_specs=pl.BlockSpec((1, 1), lambda i: (0, 0),
                               memory_space=pltpu.SMEM),
        out_shape=jax.ShapeDtypeStruct((1, 1), jnp.float32),
        scratch_shapes=[pltpu.SMEM((8,), jnp.float32),
                        pltpu.VMEM((_NROW, _R, _C2), jnp.float32),
                        pltpu.VMEM((_NROW, 128), jnp.int32)],
    )(pr, boxes, labels.astype(jnp.int32).reshape(_B, 1, _NOBJ),
      al, asr, ol, osr)
    return out[0, 0]


# speedup vs baseline: 24.8364x; 1.0102x over previous
import jax
import jax.numpy as jnp
from jax import lax
from jax.experimental import pallas as pl
from jax.experimental.pallas import tpu as pltpu

_B, _P, _NOBJ, _NC = 16, 16320, 12, 21
_R, _C = 8, 2040
_THRESHOLD, _NEG_POS_RATIO, _THETA, _ALPHA = 0.5, 3, 0.01, 1.0


def _flat_idx():
    r = lax.broadcasted_iota(jnp.int32, (_R, _C), 0)
    c = lax.broadcasted_iota(jnp.int32, (_R, _C), 1)
    return r * _C + c


def _match(boxes_ref, labels_ref, ax1, ay1, ax2, ay2, pcx, pcy, pw, ph):
    area_b = (ax2 - ax1) * (ay2 - ay1)
    fidx = _flat_idx()
    best = None
    obj = None
    mxs, pfs = [], []
    for j in range(_NOBJ):
        bx1 = boxes_ref[0, j, 0]
        by1 = boxes_ref[0, j, 1]
        bx2 = boxes_ref[0, j, 2]
        by2 = boxes_ref[0, j, 3]
        w = jnp.maximum(jnp.minimum(bx2, ax2) - jnp.maximum(bx1, ax1), 0.0)
        h = jnp.maximum(jnp.minimum(by2, ay2) - jnp.maximum(by1, ay1), 0.0)
        inter = w * h
        area_a = (bx2 - bx1) * (by2 - by1)
        iou = inter / (area_a + area_b - inter)
        mx = jnp.max(iou, axis=(0, 1), keepdims=True)
        pfs.append(jnp.min(jnp.where(iou == mx, fidx, _P), axis=(0, 1),
                           keepdims=True))
        mxs.append(mx)
        if j == 0:
            best = iou
            obj = jnp.zeros((_R, _C), jnp.int32)
        else:
            obj = jnp.where(iou > best, j, obj)
            best = jnp.maximum(best, iou)
    rank = jnp.zeros((1, 1), jnp.int32) - 1
    force = jnp.zeros((_R, _C), jnp.int32)
    for j in range(_NOBJ):
        valid = mxs[j] > 0.0
        rank = rank + valid.astype(jnp.int32)
        sv = jnp.where(valid, rank + 1, 0)
        force = jnp.maximum(force, jnp.where(fidx == pfs[j], sv, 0))
    forced = force > 0
    best = jnp.where(forced, 1.0, best)
    obj = jnp.where(forced, force - 1, obj)
    lab = jnp.zeros((_R, _C), jnp.int32)
    gx1 = jnp.zeros((_R, _C), jnp.float32)
    gy1 = jnp.zeros((_R, _C), jnp.float32)
    gx2 = jnp.zeros((_R, _C), jnp.float32)
    gy2 = jnp.zeros((_R, _C), jnp.float32)
    for j in range(_NOBJ):
        sel = obj == j
        lab = jnp.where(sel, labels_ref[0, 0, j], lab)
        gx1 = jnp.where(sel, boxes_ref[0, j, 0], gx1)
        gy1 = jnp.where(sel, boxes_ref[0, j, 1], gy1)
        gx2 = jnp.where(sel, boxes_ref[0, j, 2], gx2)
        gy2 = jnp.where(sel, boxes_ref[0, j, 3], gy2)
    lab = jnp.where(best < _THRESHOLD, 0, lab)
    cx = (gx1 + gx2) / 2.0
    cy = (gy1 + gy2) / 2.0
    w = gx2 - gx1
    h = gy2 - gy1
    t0 = (cx - pcx) / (pw / 10.0)
    t1 = (cy - pcy) / (ph / 10.0)
    t2 = jnp.log(w / pw) * 5.0
    t3 = jnp.log(h / ph) * 5.0
    return lab, (t0, t1, t2, t3)


_C2 = 2048
_NROW = 2 * _B


def _batched_topk_sums(neg_ref, kv_ref):
    kcol = kv_ref[:, 0:1]

    def count_ge(mid):
        cnt = jnp.zeros((_NROW, 128), jnp.int32)
        m3 = mid[:, None, :]
        for c in range(_C2 // 128):
            blk = lax.bitcast_convert_type(
                neg_ref[:, :, pl.ds(c * 128, 128)], jnp.int32)
            cnt = cnt + jnp.sum((blk >= m3).astype(jnp.int32), axis=1)
        return jnp.sum(cnt, axis=1, keepdims=True)

    def body(_, lohi):
        lo, hi = lohi
        mid = lo + (hi - lo + 1) // 2
        ok = count_ge(mid) >= kcol
        return jnp.where(ok, mid, lo), jnp.where(ok, hi, mid - 1)

    lo, _ = lax.fori_loop(
        0, 31, body,
        (jnp.zeros((_NROW, 1), jnp.int32),
         jnp.full((_NROW, 1), 0x7F800000, jnp.int32)))
    tf = lax.bitcast_convert_type(lo, jnp.float32)
    srel = jnp.zeros((_NROW, 128), jnp.float32)
    t3 = tf[:, None, :]
    for c in range(_C2 // 128):
        blk = neg_ref[:, :, pl.ds(c * 128, 128)]
        srel = srel + jnp.sum(jnp.maximum(blk - t3, 0.0), axis=1)
    s = jnp.sum(srel, axis=1, keepdims=True)
    kf = kcol.astype(jnp.float32)
    hard = jnp.where(kcol > 0, kf * tf + s, 0.0)
    return jnp.sum(hard[:_B, 0]), jnp.sum(hard[_B:, 0])


def _loc_loss_sum(pred, tgt, posf):
    acc = jnp.float32(0.0)
    for c in range(4):
        d = jnp.abs(pred[c] - tgt[c])
        acc = acc + jnp.sum(jnp.where(d < 1.0, 0.5 * d * d, d - 0.5) * posf)
    return acc


def _body(pr_ref, boxes_ref, labels_ref, al_ref, as_ref, ol_ref, os_ref,
          out_ref, acc_ref, neg_ref, kv_ref):
    i = pl.program_id(0)

    @pl.when(i == 0)
    def _zero():
        neg_ref[...] = jnp.zeros((_NROW, _R, _C2), jnp.float32)

    pcx = pr_ref[0]
    pcy = pr_ref[1]
    pw = pr_ref[2]
    ph = pr_ref[3]
    px1 = pcx - pw / 2.0
    py1 = pcy - ph / 2.0
    px2 = pcx + pw / 2.0
    py2 = pcy + ph / 2.0

    lab_a, ta = _match(boxes_ref, labels_ref, px1, py1, px2, py2,
                       pcx, pcy, pw, ph)
    pos_a = lab_a > 0
    posf_a = pos_a.astype(jnp.float32)
    n_pos_a = jnp.sum(posf_a)
    al = [al_ref[0, c] for c in range(4)]
    loc_a = _loc_loss_sum(al, ta, posf_a)
    s0 = as_ref[0, 0]
    s1 = as_ref[0, 1]
    m2 = jnp.maximum(s0, s1)
    lse2 = m2 + jnp.log(jnp.exp(s0 - m2) + jnp.exp(s1 - m2))
    ce_a = lse2 - jnp.where(pos_a, s1, s0)
    cpos_a = jnp.sum(ce_a * posf_a)
    neg_a = jnp.where(pos_a, 0.0, ce_a)
    k_a = _NEG_POS_RATIO * jnp.sum(pos_a.astype(jnp.int32))
    neg_ref[i, :, pl.ds(0, _C)] = neg_a
    kv_ref[pl.ds(i, 1), :] = jnp.full((1, 128), k_a, jnp.int32)

    a0 = al[0]
    a1 = al[1]
    a2 = al[2]
    a3 = al[3]
    dcx = a0 * pw / 10.0 + pcx
    dcy = a1 * ph / 10.0 + pcy
    dw = jnp.exp(a2 / 5.0) * pw
    dh = jnp.exp(a3 / 5.0) * ph
    dx1 = dcx - dw / 2.0
    dy1 = dcy - dh / 2.0
    dx2 = dcx + dw / 2.0
    dy2 = dcy + dh / 2.0
    ocx = (dx2 + dx1) / 2.0
    ocy = (dy2 + dy1) / 2.0
    ow = dx2 - dx1
    oh = dy2 - dy1
    lab_o, to = _match(boxes_ref, labels_ref, dx1, dy1, dx2, dy2,
                       ocx, ocy, ow, oh)
    em = jnp.maximum(s0, s1)
    e0 = jnp.exp(s0 - em)
    e1 = jnp.exp(s1 - em)
    easy = e1 / (e0 + e1) < _THETA
    pos_o = jnp.logical_and(lab_o > 0, jnp.logical_not(easy))
    posf_o = pos_o.astype(jnp.float32)
    n_pos_o = jnp.sum(posf_o)
    ol = [ol_ref[0, c] for c in range(4)]
    loc_o = _loc_loss_sum(ol, to, posf_o)
    sc = [os_ref[0, c] for c in range(_NC)]
    mo = sc[0]
    for c in range(1, _NC):
        mo = jnp.maximum(mo, sc[c])
    se = jnp.zeros((_R, _C), jnp.float32)
    st = jnp.zeros((_R, _C), jnp.float32)
    for c in range(_NC):
        se = se + jnp.exp(sc[c] - mo)
        st = st + jnp.where(lab_o == c, sc[c], 0.0)
    ce_o = (mo + jnp.log(se)) - st
    cpos_o = jnp.sum(ce_o * posf_o)
    neg_o = jnp.where(pos_o, 0.0, ce_o)
    neg_o = jnp.where(easy, 0.0, neg_o)
    k_o = _NEG_POS_RATIO * jnp.sum(pos_o.astype(jnp.int32))
    neg_ref[_B + i, :, pl.ds(0, _C)] = neg_o
    kv_ref[pl.ds(_B + i, 1), :] = jnp.full((1, 128), k_o, jnp.int32)

    @pl.when(i == 0)
    def _init():
        for t in range(6):
            acc_ref[t] = 0.0

    parts = (loc_a, cpos_a, n_pos_a, loc_o, cpos_o, n_pos_o)
    for t, v in enumerate(parts):
        acc_ref[t] = acc_ref[t] + v

    @pl.when(i == _B - 1)
    def _fin():
        hard_a, hard_o = _batched_topk_sums(neg_ref, kv_ref)
        na = acc_ref[2]
        no = acc_ref[5]
        arm = (hard_a + acc_ref[1]) / na + _ALPHA * acc_ref[0] / (na * 4.0)
        odm = (hard_o + acc_ref[4]) / no + _ALPHA * acc_ref[3] / (no * 4.0)
        out_ref[0, 0] = arm + odm


def kernel(arm_locs, arm_scores, odm_locs, odm_scores, boxes, labels,
           priors_cxcy):
    al = arm_locs.transpose(0, 2, 1).reshape(_B, 4, _R, _C)
    asr = arm_scores.transpose(0, 2, 1).reshape(_B, 2, _R, _C)
    ol = odm_locs.transpose(0, 2, 1).reshape(_B, 4, _R, _C)
    osr = odm_scores.transpose(0, 2, 1).reshape(_B, _NC, _R, _C)
    pr = priors_cxcy.T.reshape(4, _R, _C)
    out = pl.pallas_call(
        _body,
        grid=(_B,),
        in_specs=[
            pl.BlockSpec((4, _R, _C), lambda i: (0, 0, 0)),
            pl.BlockSpec((1, _NOBJ, 4), lambda i: (i, 0, 0),
                         memory_space=pltpu.SMEM),
            pl.BlockSpec((1, 1, _NOBJ), lambda i: (i, 0, 0),
                         memory_space=pltpu.SMEM),
            pl.BlockSpec((1, 4, _R, _C), lambda i: (i, 0, 0, 0)),
            pl.BlockSpec((1, 2, _R, _C), lambda i: (i, 0, 0, 0)),
            pl.BlockSpec((1, 4, _R, _C), lambda i: (i, 0, 0, 0)),
            pl.BlockSpec((1, _NC, _R, _C), lambda i: (i, 0, 0, 0)),
        ],
        out_specs=pl.BlockSpec((1, 1), lambda i: (0, 0),
                               memory_space=pltpu.SMEM),
        out_shape=jax.ShapeDtypeStruct((1, 1), jnp.float32),
        scratch_shapes=[pltpu.SMEM((8,), jnp.float32),
                        pltpu.VMEM((_NROW, _R, _C2), jnp.float32),
                        pltpu.VMEM((_NROW, 128), jnp.int32)],
    )(pr, boxes, labels.astype(jnp.int32).reshape(_B, 1, _NOBJ),
      al, asr, ol, osr)
    return out[0, 0]
